# baseline (speedup 1.0000x reference)
import functools

import jax
import jax.numpy as jnp
from jax import lax
from jax.experimental import pallas as pl
from jax.experimental.pallas import tpu as pltpu
from jax.experimental.pallas import tpu_sc as plsc

_N = 10000
_E = 320000
_H = 128
_DEPTH = 4
_MAX_IN_DEG = 10
_XP = 16
_C = 128
_NC = 2
_NS = 16
_NW = _NC * _NS
_BE = 2000
_BN = 2000
_EH = _E // 2

_f32 = jnp.float32


def _silu(v):
    return v * (1.0 / (1.0 + jnp.exp(-v)))


def _mesh():
    return plsc.VectorSubcoreMesh(core_axis_name="c", subcore_axis_name="s",
                                  num_cores=_NC, num_subcores=_NS)


def _sc_gather_h(h, src, dst, nchunks):
    ne = nchunks * _C
    out_type = (
        jax.ShapeDtypeStruct((ne, _H), _f32),
        jax.ShapeDtypeStruct((ne, _H), _f32),
    )
    scratch = [
        pltpu.VMEM((_C,), jnp.int32),
        pltpu.VMEM((_C,), jnp.int32),
        pltpu.VMEM((_C,), jnp.int32),
        pltpu.VMEM((_C,), jnp.int32),
        pltpu.VMEM((_C, _H), _f32),
        pltpu.VMEM((_C, _H), _f32),
        pltpu.VMEM((_C, _H), _f32),
        pltpu.VMEM((_C, _H), _f32),
        pltpu.SemaphoreType.DMA,
        pltpu.SemaphoreType.DMA,
        pltpu.SemaphoreType.DMA,
        pltpu.SemaphoreType.DMA,
    ]

    def body(h_h, src_h, dst_h, hd_h, hs_h,
             idx_dA, idx_sA, idx_dB, idx_sB, bufdA, bufsA, bufdB, bufsB,
             s1A, s2A, s1B, s2B):
        c = lax.axis_index("c")
        s = lax.axis_index("s")
        wid = s * _NC + c

        def fire(j, idx_d, idx_s, bufd, bufs, e1, e2):
            base = j * _C
            pltpu.sync_copy(dst_h.at[pl.ds(base, _C)], idx_d)
            pltpu.sync_copy(src_h.at[pl.ds(base, _C)], idx_s)
            pltpu.async_copy(h_h.at[idx_d], bufd, e1)
            pltpu.async_copy(h_h.at[idx_s], bufs, e2)

        def drain_write(j, idx_d, idx_s, bufd, bufs, e1, e2):
            pltpu.make_async_copy(h_h.at[idx_d], bufd, e1).wait()
            pltpu.make_async_copy(h_h.at[idx_s], bufs, e2).wait()
            base = j * _C
            pltpu.sync_copy(bufd, hd_h.at[pl.ds(base, _C)])
            pltpu.sync_copy(bufs, hs_h.at[pl.ds(base, _C)])

        A = (idx_dA, idx_sA, bufdA, bufsA, s1A, s2A)
        B = (idx_dB, idx_sB, bufdB, bufsB, s1B, s2B)

        base = nchunks // _NW
        extra = nchunks - base * _NW
        npip = base - (base % 2)
        npair = npip // 2

        fire(wid, *A)

        def step(g, carry):
            c1 = wid + _NW * (2 * g + 1)
            fire(c1, *B)
            drain_write(wid + _NW * (2 * g), *A)
            fire(wid + _NW * (2 * g + 2), *A)
            drain_write(c1, *B)
            return carry

        lax.fori_loop(0, npair - 1, step, 0)

        c1 = wid + _NW * (npip - 1)
        fire(c1, *B)
        drain_write(wid + _NW * (npip - 2), *A)
        drain_write(c1, *B)

        for i in range(npip, base):
            fire(wid + _NW * i, *A)
            drain_write(wid + _NW * i, *A)

        @pl.when(wid < extra)
        def _():
            j = base * _NW + wid
            fire(j, *A)
            drain_write(j, *A)

    return pl.kernel(body, out_type=out_type, mesh=_mesh(), scratch_types=scratch,
                     compiler_params=pltpu.CompilerParams(use_tc_tiling_on_sc=True))(
        h, src, dst)


def _sc_gather_x(xpad, src, dst):
    nchunks = _E // _C
    out_type = (
        jax.ShapeDtypeStruct((_E, _XP), _f32),
        jax.ShapeDtypeStruct((_E, _XP), _f32),
    )
    scratch = [
        pltpu.VMEM((_C,), jnp.int32),
        pltpu.VMEM((_C,), jnp.int32),
        pltpu.VMEM((_C,), jnp.int32),
        pltpu.VMEM((_C,), jnp.int32),
        pltpu.VMEM((_C, _XP), _f32),
        pltpu.VMEM((_C, _XP), _f32),
        pltpu.VMEM((_C, _XP), _f32),
        pltpu.VMEM((_C, _XP), _f32),
        pltpu.SemaphoreType.DMA,
        pltpu.SemaphoreType.DMA,
        pltpu.SemaphoreType.DMA,
        pltpu.SemaphoreType.DMA,
    ]

    def body(x_h, src_h, dst_h, xs_h, xd_h,
             idx_dA, idx_sA, idx_dB, idx_sB, bufdA, bufsA, bufdB, bufsB,
             s1A, s2A, s1B, s2B):
        c = lax.axis_index("c")
        s = lax.axis_index("s")
        wid = s * _NC + c

        def fire(j, idx_d, idx_s, bufd, bufs, e1, e2):
            base = j * _C
            pltpu.sync_copy(dst_h.at[pl.ds(base, _C)], idx_d)
            pltpu.sync_copy(src_h.at[pl.ds(base, _C)], idx_s)
            pltpu.async_copy(x_h.at[idx_d], bufd, e1)
            pltpu.async_copy(x_h.at[idx_s], bufs, e2)

        def drain_write(j, idx_d, idx_s, bufd, bufs, e1, e2):
            pltpu.make_async_copy(x_h.at[idx_d], bufd, e1).wait()
            pltpu.make_async_copy(x_h.at[idx_s], bufs, e2).wait()
            base = j * _C
            pltpu.sync_copy(bufd, xd_h.at[pl.ds(base, _C)])
            pltpu.sync_copy(bufs, xs_h.at[pl.ds(base, _C)])

        A = (idx_dA, idx_sA, bufdA, bufsA, s1A, s2A)
        B = (idx_dB, idx_sB, bufdB, bufsB, s1B, s2B)

        base = nchunks // _NW
        extra = nchunks - base * _NW
        npip = base - (base % 2)
        npair = npip // 2

        fire(wid, *A)

        def step(g, carry):
            c1 = wid + _NW * (2 * g + 1)
            fire(c1, *B)
            drain_write(wid + _NW * (2 * g), *A)
            fire(wid + _NW * (2 * g + 2), *A)
            drain_write(c1, *B)
            return carry

        lax.fori_loop(0, npair - 1, step, 0)

        c1 = wid + _NW * (npip - 1)
        fire(c1, *B)
        drain_write(wid + _NW * (npip - 2), *A)
        drain_write(c1, *B)

        for i in range(npip, base):
            fire(wid + _NW * i, *A)
            drain_write(wid + _NW * i, *A)

        @pl.when(wid < extra)
        def _():
            j = base * _NW + wid
            fire(j, *A)
            drain_write(j, *A)

    return pl.kernel(body, out_type=out_type, mesh=_mesh(), scratch_types=scratch,
                     compiler_params=pltpu.CompilerParams(use_tc_tiling_on_sc=False))(
        xpad, src, dst)


def _sc_scatter_m(m, dst, nchunks):
    np_rows = 10240
    out_type = jax.ShapeDtypeStruct((_NC, np_rows, _H), _f32)
    scratch = [
        pltpu.VMEM((_C,), jnp.int32),
        pltpu.VMEM((_C,), jnp.int32),
        pltpu.VMEM((_C, _H), _f32),
        pltpu.VMEM((_C, _H), _f32),
        pltpu.VMEM_SHARED((10240, _H), _f32),
        pltpu.SemaphoreType.DMA,
        pltpu.SemaphoreType.DMA,
        pltpu.SemaphoreType.DMA,
        pltpu.SemaphoreType.DMA,
    ]
    rpt = 10240 // _NS
    zc = 128

    def body(m_h, dst_h, agg_h, idxA, idxB, bufA, bufB, shm, siA, smA, siB, smB):
        c = lax.axis_index("c")
        s = lax.axis_index("s")

        def zm(t, carry):
            r = t // (_H // 16)
            k = t % (_H // 16)
            bufA[r, pl.ds(k * 16, 16)] = jnp.zeros((16,), _f32)
            return carry

        lax.fori_loop(0, _C * (_H // 16), zm, 0)
        for r in range(rpt // zc):
            pltpu.sync_copy(bufA.at[pl.ds(0, zc)],
                            shm.at[pl.ds(s * rpt + r * zc, zc)])
        plsc.subcore_barrier()

        def chunk(i):
            return (c + _NC * (s + _NS * i)) * _C

        def load(i, idx, buf, si, sm):
            base = chunk(i)
            pltpu.async_copy(dst_h.at[pl.ds(base, _C)], idx, si)
            pltpu.async_copy(m_h.at[pl.ds(base, _C)], buf, sm)

        def scat(i, idx, buf, si, sm):
            base = chunk(i)
            pltpu.make_async_copy(dst_h.at[pl.ds(base, _C)], idx, si).wait()
            pltpu.make_async_copy(m_h.at[pl.ds(base, _C)], buf, sm).wait()
            pltpu.sync_copy(buf, shm.at[idx], add=True)

        A = (idxA, bufA, siA, smA)
        B = (idxB, bufB, siB, smB)

        percore = nchunks // _NC
        base_t = percore // _NS
        extra_t = percore - base_t * _NS
        npip = base_t - (base_t % 2)
        npair = npip // 2

        load(0, *A)

        def step(g, carry):
            load(2 * g + 1, *B)
            scat(2 * g, *A)
            load(2 * g + 2, *A)
            scat(2 * g + 1, *B)
            return carry

        lax.fori_loop(0, npair - 1, step, 0)
        load(npip - 1, *B)
        scat(npip - 2, *A)
        scat(npip - 1, *B)

        for i in range(npip, base_t):
            load(i, *A)
            scat(i, *A)

        @pl.when(s < extra_t)
        def _():
            load(base_t, *A)
            scat(base_t, *A)

        plsc.subcore_barrier()
        pltpu.sync_copy(shm.at[pl.ds(s * rpt, rpt)],
                        agg_h.at[c, pl.ds(s * rpt, rpt)])

    return pl.kernel(body, out_type=out_type, mesh=_mesh(), scratch_types=scratch,
                     compiler_params=pltpu.CompilerParams(use_tc_tiling_on_sc=True))(
        m, dst)


def _sc_scatter_v(v1, v2, dst1, dst2):
    out_type = jax.ShapeDtypeStruct((_NC, _N, _XP), _f32)
    scratch = [
        pltpu.VMEM((_C,), jnp.int32),
        pltpu.VMEM((_C,), jnp.int32),
        pltpu.VMEM((_C, _XP), _f32),
        pltpu.VMEM((_C, _XP), _f32),
        pltpu.VMEM_SHARED((_N, _XP), _f32),
        pltpu.SemaphoreType.DMA,
        pltpu.SemaphoreType.DMA,
        pltpu.SemaphoreType.DMA,
        pltpu.SemaphoreType.DMA,
    ]
    rpt = _N // _NS
    zc = 125
    nchunks = _EH // _C

    def body(v1_h, v2_h, d1_h, d2_h, agg_h, idxA, idxB, bufA, bufB, shx,
             siA, smA, siB, smB):
        c = lax.axis_index("c")
        s = lax.axis_index("s")

        def zv(t, carry):
            bufA[t, :] = jnp.zeros((_XP,), _f32)
            return carry

        lax.fori_loop(0, _C, zv, 0)
        for r in range(rpt // zc):
            pltpu.sync_copy(bufA.at[pl.ds(0, zc)],
                            shx.at[pl.ds(s * rpt + r * zc, zc)])
        plsc.subcore_barrier()

        base_t = nchunks // _NS
        extra_t = nchunks - base_t * _NS
        npip = base_t - (base_t % 2)
        npair = npip // 2

        def make(v_h, d_h):
            def chunk(i):
                return (s + _NS * i) * _C

            def load(i, idx, buf, si, sm):
                base = chunk(i)
                pltpu.async_copy(d_h.at[pl.ds(base, _C)], idx, si)
                pltpu.async_copy(v_h.at[pl.ds(base, _C)], buf, sm)

            def scat(i, idx, buf, si, sm):
                base = chunk(i)
                pltpu.make_async_copy(d_h.at[pl.ds(base, _C)], idx, si).wait()
                pltpu.make_async_copy(v_h.at[pl.ds(base, _C)], buf, sm).wait()
                pltpu.sync_copy(buf, shx.at[idx], add=True)

            def run():
                A = (idxA, bufA, siA, smA)
                B = (idxB, bufB, siB, smB)
                load(0, *A)

                def step(g, carry):
                    load(2 * g + 1, *B)
                    scat(2 * g, *A)
                    load(2 * g + 2, *A)
                    scat(2 * g + 1, *B)
                    return carry

                lax.fori_loop(0, npair - 1, step, 0)
                load(npip - 1, *B)
                scat(npip - 2, *A)
                scat(npip - 1, *B)
                for i in range(npip, base_t):
                    load(i, *A)
                    scat(i, *A)

                @pl.when(s < extra_t)
                def _():
                    load(base_t, *A)
                    scat(base_t, *A)

            return run

        pl.when(c == 0)(make(v1_h, d1_h))
        pl.when(c == 1)(make(v2_h, d2_h))

        plsc.subcore_barrier()
        pltpu.sync_copy(shx.at[pl.ds(s * rpt, rpt)],
                        agg_h.at[c, pl.ds(s * rpt, rpt)])

    return pl.kernel(body, out_type=out_type, mesh=_mesh(), scratch_types=scratch,
                     compiler_params=pltpu.CompilerParams(use_tc_tiling_on_sc=False))(
        v1, v2, dst1, dst2)


def _full2(shape):
    return pl.BlockSpec(shape, lambda i: (0, 0))


def _tc_embed(feat, Win, b_in):
    def body(f_r, win_r, bin_r, h_r):
        h_r[...] = jnp.dot(f_r[...], win_r[...], preferred_element_type=_f32) + bin_r[...]

    row = pl.BlockSpec((_BN, _H), lambda i: (i, 0))
    return pl.pallas_call(
        body,
        grid=(_N // _BN,),
        in_specs=[row, _full2((_H, _H)), _full2((1, _H))],
        out_specs=row,
        out_shape=jax.ShapeDtypeStruct((_N, _H), _f32),
    )(feat, Win, b_in.reshape(1, _H))


def _tc_edge(hd, hs, xs, xd, half, We1l, be1l, We2l, be2l, Wc1l, bc1l,
             Wc2l, bc2l):
    def body(hd_r, hs_r, xs_r, xd_r, we1_r, be1_r, we2_r, be2_r, wc1_r, bc1_r,
             wc2_r, bc2_r, m_r, v_r):
        diff = xd_r[...] - xs_r[...]
        r2 = jnp.sum(diff * diff, axis=-1, keepdims=True)
        em = jnp.concatenate([hd_r[...], hs_r[...], r2], axis=-1)
        u = _silu(jnp.dot(em, we1_r[...], preferred_element_type=_f32) + be1_r[...])
        m = _silu(jnp.dot(u, we2_r[...], preferred_element_type=_f32) + be2_r[...])
        t = _silu(jnp.dot(m, wc1_r[...], preferred_element_type=_f32) + bc1_r[...])
        cw = jnp.dot(t, wc2_r[...], preferred_element_type=_f32) + bc2_r[...]
        m_r[...] = m
        v_r[...] = diff * cw

    off = half * (_EH // _BE)
    erow = pl.BlockSpec((_BE, _H), lambda i: (i, 0))
    xoff = pl.BlockSpec((_BE, _XP), lambda i: (i + off, 0))
    xrow = pl.BlockSpec((_BE, _XP), lambda i: (i, 0))
    return pl.pallas_call(
        body,
        grid=(_EH // _BE,),
        in_specs=[erow, erow, xoff, xoff, _full2((2 * _H + 1, _H)),
                  _full2((1, _H)), _full2((_H, _H)), _full2((1, _H)),
                  _full2((_H, _H)), _full2((1, _H)),
                  _full2((_H, 1)), _full2((1, 1))],
        out_specs=[erow, xrow],
        out_shape=[jax.ShapeDtypeStruct((_EH, _H), _f32),
                   jax.ShapeDtypeStruct((_EH, _XP), _f32)],
    )(hd, hs, xs, xd, We1l, be1l.reshape(1, _H), We2l, be2l.reshape(1, _H),
      Wc1l, bc1l.reshape(1, _H), Wc2l, bc2l.reshape(1, 1))


def _tc_node(h, x, ag1m, ag2m, agv, Wn1l, bn1l, Wn2l, bn2l):
    def body(h_r, x_r, a1m_r, a2m_r, av_r, wn1_r, bn1_r, wn2_r, bn2_r,
             h2_r, x2_r):
        am = a1m_r[0] + a1m_r[1] + a2m_r[0] + a2m_r[1]
        ax = av_r[0] + av_r[1]
        nm = jnp.concatenate([h_r[...], am], axis=-1)
        g = _silu(jnp.dot(nm, wn1_r[...], preferred_element_type=_f32) + bn1_r[...])
        h2_r[...] = h_r[...] + jnp.dot(g, wn2_r[...], preferred_element_type=_f32) + bn2_r[...]
        x2_r[...] = x_r[...] + ax / _MAX_IN_DEG

    row = pl.BlockSpec((_BN, _H), lambda i: (i, 0))
    xrow = pl.BlockSpec((_BN, _XP), lambda i: (i, 0))
    amrow = pl.BlockSpec((_NC, _BN, _H), lambda i: (0, i, 0))
    axrow = pl.BlockSpec((_NC, _BN, _XP), lambda i: (0, i, 0))
    return pl.pallas_call(
        body,
        grid=(_N // _BN,),
        in_specs=[row, xrow, amrow, amrow, axrow, _full2((2 * _H, _H)),
                  _full2((1, _H)), _full2((_H, _H)), _full2((1, _H))],
        out_specs=[row, xrow],
        out_shape=[jax.ShapeDtypeStruct((_N, _H), _f32),
                   jax.ShapeDtypeStruct((_N, _XP), _f32)],
    )(h, x, ag1m, ag2m, agv, Wn1l, bn1l.reshape(1, _H), Wn2l,
      bn2l.reshape(1, _H))


def _tc_node_last(h, x, ag1m, ag2m, agv, Wn1l, bn1l, Wn2l, bn2l, Wout, b_out):
    def body(h_r, x_r, a1m_r, a2m_r, av_r, wn1_r, bn1_r, wn2_r, bn2_r,
             wo_r, bo_r, o_r, x2_r):
        am = a1m_r[0] + a1m_r[1] + a2m_r[0] + a2m_r[1]
        ax = av_r[0] + av_r[1]
        nm = jnp.concatenate([h_r[...], am], axis=-1)
        g = _silu(jnp.dot(nm, wn1_r[...], preferred_element_type=_f32) + bn1_r[...])
        h2 = h_r[...] + jnp.dot(g, wn2_r[...], preferred_element_type=_f32) + bn2_r[...]
        o_r[...] = jnp.dot(h2, wo_r[...], preferred_element_type=_f32) + bo_r[...]
        x2_r[...] = x_r[...] + ax / _MAX_IN_DEG

    row = pl.BlockSpec((_BN, _H), lambda i: (i, 0))
    xrow = pl.BlockSpec((_BN, _XP), lambda i: (i, 0))
    amrow = pl.BlockSpec((_NC, _BN, _H), lambda i: (0, i, 0))
    axrow = pl.BlockSpec((_NC, _BN, _XP), lambda i: (0, i, 0))
    return pl.pallas_call(
        body,
        grid=(_N // _BN,),
        in_specs=[row, xrow, amrow, amrow, axrow, _full2((2 * _H, _H)),
                  _full2((1, _H)), _full2((_H, _H)), _full2((1, _H)),
                  _full2((_H, _H)), _full2((1, _H))],
        out_specs=[row, xrow],
        out_shape=[jax.ShapeDtypeStruct((_N, _H), _f32),
                   jax.ShapeDtypeStruct((_N, _XP), _f32)],
    )(h, x, ag1m, ag2m, agv, Wn1l, bn1l.reshape(1, _H), Wn2l,
      bn2l.reshape(1, _H), Wout, b_out.reshape(1, _H))


def kernel(feat, coordinate, edge_index, Win, b_in, Wout, b_out,
           We1, be1, We2, be2, Wc1, bc1, Wc2, bc2, Wn1, bn1, Wn2, bn2):
    src = edge_index[0]
    dst = edge_index[1]
    src1, src2 = src[:_EH], src[_EH:]
    dst1, dst2 = dst[:_EH], dst[_EH:]
    x = jnp.pad(coordinate, ((0, 0), (0, _XP - 3)))
    nch = _EH // _C

    h = _tc_embed(feat, Win, b_in)
    out = None
    for l in range(_DEPTH):
        wl = (We1[l], be1[l], We2[l], be2[l], Wc1[l], bc1[l], Wc2[l], bc2[l])
        xs, xd = _sc_gather_x(x, src, dst)
        hd1, hs1 = _sc_gather_h(h, src1, dst1, nch)
        m1, v1 = _tc_edge(hd1, hs1, xs, xd, 0, *wl)
        hd2, hs2 = _sc_gather_h(h, src2, dst2, nch)
        ag1m = _sc_scatter_m(m1, dst1, nch)
        m2, v2 = _tc_edge(hd2, hs2, xs, xd, 1, *wl)
        ag2m = _sc_scatter_m(m2, dst2, nch)
        agv = _sc_scatter_v(v1, v2, dst1, dst2)
        if l < _DEPTH - 1:
            h, x = _tc_node(h, x, ag1m, ag2m, agv,
                            Wn1[l], bn1[l], Wn2[l], bn2[l])
        else:
            out, x = _tc_node_last(h, x, ag1m, ag2m, agv,
                                   Wn1[l], bn1[l], Wn2[l], bn2[l],
                                   Wout, b_out)
    return (out, x[:, :3])


# speedup vs baseline: 1.0968x; 1.0968x over previous
import functools

import jax
import jax.numpy as jnp
from jax import lax
from jax.experimental import pallas as pl
from jax.experimental.pallas import tpu as pltpu
from jax.experimental.pallas import tpu_sc as plsc

_N = 10000
_E = 320000
_H = 128
_DEPTH = 4
_MAX_IN_DEG = 10
_XP = 16
_C = 128
_NCHUNKS = _E // _C
_NPAIR = 39
_NC = 2
_NS = 16
_NW = _NC * _NS
_BE = 2000
_BN = 2000

_f32 = jnp.float32


def _silu(v):
    return v * (1.0 / (1.0 + jnp.exp(-v)))


def _sc_gather(h, xpad, src, dst, nchunks):
    ne = nchunks * _C
    mesh = plsc.VectorSubcoreMesh(core_axis_name="c", subcore_axis_name="s",
                                  num_cores=_NC, num_subcores=_NS)
    out_type = (
        jax.ShapeDtypeStruct((ne, _H), _f32),
        jax.ShapeDtypeStruct((ne, _H), _f32),
        jax.ShapeDtypeStruct((ne, _XP), _f32),
        jax.ShapeDtypeStruct((ne, _XP), _f32),
    )
    scratch = [
        pltpu.VMEM((_C,), jnp.int32),
        pltpu.VMEM((_C,), jnp.int32),
        pltpu.VMEM((_C,), jnp.int32),
        pltpu.VMEM((_C,), jnp.int32),
        pltpu.VMEM((_C, _H), _f32),
        pltpu.VMEM((_C, _H), _f32),
        pltpu.VMEM((_C, _XP), _f32),
        pltpu.VMEM((_C, _XP), _f32),
        pltpu.VMEM((_C, _H), _f32),
        pltpu.VMEM((_C, _H), _f32),
        pltpu.VMEM((_C, _XP), _f32),
        pltpu.VMEM((_C, _XP), _f32),
        pltpu.SemaphoreType.DMA,
        pltpu.SemaphoreType.DMA,
        pltpu.SemaphoreType.DMA,
        pltpu.SemaphoreType.DMA,
        pltpu.SemaphoreType.DMA,
        pltpu.SemaphoreType.DMA,
        pltpu.SemaphoreType.DMA,
        pltpu.SemaphoreType.DMA,
    ]

    def body(h_h, x_h, src_h, dst_h, hd_h, hs_h, xs_h, xd_h,
             idx_dA, idx_sA, idx_dB, idx_sB,
             bufdA, bufsA, bufxsA, bufxdA, bufdB, bufsB, bufxsB, bufxdB,
             s1A, s2A, s3A, s4A, s1B, s2B, s3B, s4B):
        c = lax.axis_index("c")
        s = lax.axis_index("s")
        wid = s * _NC + c

        def load_idx(j, idx_d, idx_s):
            base = j * _C
            pltpu.sync_copy(dst_h.at[pl.ds(base, _C)], idx_d)
            pltpu.sync_copy(src_h.at[pl.ds(base, _C)], idx_s)

        def start(idx_d, idx_s, bufd, bufs, bufxs, bufxd, e1, e2, e3, e4):
            pltpu.async_copy(h_h.at[idx_d], bufd, e1)
            pltpu.async_copy(h_h.at[idx_s], bufs, e2)
            pltpu.async_copy(x_h.at[idx_s], bufxs, e3)
            pltpu.async_copy(x_h.at[idx_d], bufxd, e4)

        def wait(idx_d, idx_s, bufd, bufs, bufxs, bufxd, e1, e2, e3, e4):
            pltpu.make_async_copy(h_h.at[idx_d], bufd, e1).wait()
            pltpu.make_async_copy(h_h.at[idx_s], bufs, e2).wait()
            pltpu.make_async_copy(x_h.at[idx_s], bufxs, e3).wait()
            pltpu.make_async_copy(x_h.at[idx_d], bufxd, e4).wait()

        def write(j, bufd, bufs, bufxs, bufxd):
            base = j * _C
            pltpu.sync_copy(bufd, hd_h.at[pl.ds(base, _C)])
            pltpu.sync_copy(bufs, hs_h.at[pl.ds(base, _C)])
            pltpu.sync_copy(bufxs, xs_h.at[pl.ds(base, _C)])
            pltpu.sync_copy(bufxd, xd_h.at[pl.ds(base, _C)])

        A = (idx_dA, idx_sA, bufdA, bufsA, bufxsA, bufxdA, s1A, s2A, s3A, s4A)
        B = (idx_dB, idx_sB, bufdB, bufsB, bufxsB, bufxdB, s1B, s2B, s3B, s4B)

        def fire(j, slot):
            load_idx(j, slot[0], slot[1])
            start(*slot)

        def drain_write(j, slot):
            wait(*slot)
            write(j, slot[2], slot[3], slot[4], slot[5])

        base = nchunks // _NW
        extra = nchunks - base * _NW
        npip = base - (base % 2)
        npair = npip // 2

        fire(wid, A)

        def step(g, carry):
            c1 = wid + _NW * (2 * g + 1)
            fire(c1, B)
            drain_write(wid + _NW * (2 * g), A)
            fire(wid + _NW * (2 * g + 2), A)
            drain_write(c1, B)
            return carry

        lax.fori_loop(0, npair - 1, step, 0)

        c1 = wid + _NW * (npip - 1)
        fire(c1, B)
        drain_write(wid + _NW * (npip - 2), A)
        drain_write(c1, B)

        for i in range(npip, base):
            fire(wid + _NW * i, A)
            drain_write(wid + _NW * i, A)

        @pl.when(wid < extra)
        def _():
            j = base * _NW + wid
            fire(j, A)
            drain_write(j, A)

    return pl.kernel(body, out_type=out_type, mesh=mesh, scratch_types=scratch,
                     compiler_params=pltpu.CompilerParams(use_tc_tiling_on_sc=False))(
        h, xpad, src, dst)


def _sc_scatter(m, v, dst, nchunks):
    mesh = plsc.VectorSubcoreMesh(core_axis_name="c", subcore_axis_name="s",
                                  num_cores=_NC, num_subcores=_NS)
    out_type = (
        jax.ShapeDtypeStruct((_NC, _N, _H), _f32),
        jax.ShapeDtypeStruct((_NC, _N, _XP), _f32),
    )
    scratch = [
        pltpu.VMEM((_C,), jnp.int32),
        pltpu.VMEM((_C,), jnp.int32),
        pltpu.VMEM((_C, _H), _f32),
        pltpu.VMEM((_C, _H), _f32),
        pltpu.VMEM((_C, _XP), _f32),
        pltpu.VMEM((_C, _XP), _f32),
        pltpu.VMEM_SHARED((_N, _H), _f32),
        pltpu.VMEM_SHARED((_N, _XP), _f32),
        pltpu.SemaphoreType.DMA,
        pltpu.SemaphoreType.DMA,
        pltpu.SemaphoreType.DMA,
        pltpu.SemaphoreType.DMA,
        pltpu.SemaphoreType.DMA,
        pltpu.SemaphoreType.DMA,
    ]
    rpt = _N // _NS
    zc = 125

    def body(m_h, v_h, dst_h, aggm_h, aggx_h,
             idxA, idxB, bufmA, bufmB, bufvA, bufvB, shm, shx,
             siA, smA, svA, siB, smB, svB):
        c = lax.axis_index("c")
        s = lax.axis_index("s")

        def zm(t, carry):
            r = t // (_H // 16)
            k = t % (_H // 16)
            bufmA[r, pl.ds(k * 16, 16)] = jnp.zeros((16,), _f32)
            return carry

        lax.fori_loop(0, _C * (_H // 16), zm, 0)

        def zv(t, carry):
            bufvA[t, :] = jnp.zeros((_XP,), _f32)
            return carry

        lax.fori_loop(0, _C, zv, 0)

        for r in range(rpt // zc):
            pltpu.sync_copy(bufmA.at[pl.ds(0, zc)],
                            shm.at[pl.ds(s * rpt + r * zc, zc)])
            pltpu.sync_copy(bufvA.at[pl.ds(0, zc)],
                            shx.at[pl.ds(s * rpt + r * zc, zc)])
        plsc.subcore_barrier()

        def chunk(i):
            return (c + _NC * (s + _NS * i)) * _C

        def load(i, idx, bufm, bufv, si, sm, sv):
            base = chunk(i)
            pltpu.async_copy(dst_h.at[pl.ds(base, _C)], idx, si)
            pltpu.async_copy(m_h.at[pl.ds(base, _C)], bufm, sm)
            pltpu.async_copy(v_h.at[pl.ds(base, _C)], bufv, sv)

        def scat(i, idx, bufm, bufv, si, sm, sv):
            base = chunk(i)
            pltpu.make_async_copy(dst_h.at[pl.ds(base, _C)], idx, si).wait()
            pltpu.make_async_copy(m_h.at[pl.ds(base, _C)], bufm, sm).wait()
            pltpu.make_async_copy(v_h.at[pl.ds(base, _C)], bufv, sv).wait()
            pltpu.sync_copy(bufm, shm.at[idx], add=True)
            pltpu.sync_copy(bufv, shx.at[idx], add=True)

        A = (idxA, bufmA, bufvA, siA, smA, svA)
        B = (idxB, bufmB, bufvB, siB, smB, svB)

        percore = nchunks // _NC
        base_t = percore // _NS
        extra_t = percore - base_t * _NS
        npip = base_t - (base_t % 2)
        npair = npip // 2

        load(0, *A)

        def step(g, carry):
            load(2 * g + 1, *B)
            scat(2 * g, *A)
            load(2 * g + 2, *A)
            scat(2 * g + 1, *B)
            return carry

        lax.fori_loop(0, npair - 1, step, 0)
        load(npip - 1, *B)
        scat(npip - 2, *A)
        scat(npip - 1, *B)

        for i in range(npip, base_t):
            load(i, *A)
            scat(i, *A)

        @pl.when(s < extra_t)
        def _():
            load(base_t, *A)
            scat(base_t, *A)

        plsc.subcore_barrier()
        pltpu.sync_copy(shm.at[pl.ds(s * rpt, rpt)],
                        aggm_h.at[c, pl.ds(s * rpt, rpt)])
        pltpu.sync_copy(shx.at[pl.ds(s * rpt, rpt)],
                        aggx_h.at[c, pl.ds(s * rpt, rpt)])

    return pl.kernel(body, out_type=out_type, mesh=mesh, scratch_types=scratch,
                     compiler_params=pltpu.CompilerParams(use_tc_tiling_on_sc=False))(
        m, v, dst)


def _half_edges(a):
    return a[:_E // 2], a[_E // 2:]


def _full2(shape):
    return pl.BlockSpec(shape, lambda i: (0, 0))


def _tc_embed(feat, Win, b_in):
    def body(f_r, win_r, bin_r, h_r):
        h_r[...] = jnp.dot(f_r[...], win_r[...], preferred_element_type=_f32) + bin_r[...]

    row = pl.BlockSpec((_BN, _H), lambda i: (i, 0))
    return pl.pallas_call(
        body,
        grid=(_N // _BN,),
        in_specs=[row, _full2((_H, _H)), _full2((1, _H))],
        out_specs=row,
        out_shape=jax.ShapeDtypeStruct((_N, _H), _f32),
    )(feat, Win, b_in.reshape(1, _H))


def _tc_edge(hd, hs, xs, xd, We1l, be1l, We2l, be2l, Wc1l, bc1l, Wc2l, bc2l):
    def body(hd_r, hs_r, xs_r, xd_r, we1_r, be1_r, we2_r, be2_r, wc1_r, bc1_r,
             wc2_r, bc2_r, m_r, v_r):
        diff = xd_r[...] - xs_r[...]
        r2 = jnp.sum(diff * diff, axis=-1, keepdims=True)
        em = jnp.concatenate([hd_r[...], hs_r[...], r2], axis=-1)
        u = _silu(jnp.dot(em, we1_r[...], preferred_element_type=_f32) + be1_r[...])
        m = _silu(jnp.dot(u, we2_r[...], preferred_element_type=_f32) + be2_r[...])
        t = _silu(jnp.dot(m, wc1_r[...], preferred_element_type=_f32) + bc1_r[...])
        cw = jnp.dot(t, wc2_r[...], preferred_element_type=_f32) + bc2_r[...]
        m_r[...] = m
        v_r[...] = diff * cw

    ne = hd.shape[0]
    erow = pl.BlockSpec((_BE, _H), lambda i: (i, 0))
    xrow = pl.BlockSpec((_BE, _XP), lambda i: (i, 0))
    return pl.pallas_call(
        body,
        grid=(ne // _BE,),
        in_specs=[erow, erow, xrow, xrow, _full2((2 * _H + 1, _H)),
                  _full2((1, _H)), _full2((_H, _H)), _full2((1, _H)),
                  _full2((_H, _H)), _full2((1, _H)),
                  _full2((_H, 1)), _full2((1, 1))],
        out_specs=[erow, xrow],
        out_shape=[jax.ShapeDtypeStruct((ne, _H), _f32),
                   jax.ShapeDtypeStruct((ne, _XP), _f32)],
    )(hd, hs, xs, xd, We1l, be1l.reshape(1, _H), We2l, be2l.reshape(1, _H),
      Wc1l, bc1l.reshape(1, _H), Wc2l, bc2l.reshape(1, 1))


def _tc_node(h, x, ag1m, ag1x, ag2m, ag2x, Wn1l, bn1l, Wn2l, bn2l):
    def body(h_r, x_r, a1m_r, a1x_r, a2m_r, a2x_r,
             wn1_r, bn1_r, wn2_r, bn2_r, h2_r, x2_r):
        am = a1m_r[0] + a1m_r[1] + a2m_r[0] + a2m_r[1]
        ax = a1x_r[0] + a1x_r[1] + a2x_r[0] + a2x_r[1]
        nm = jnp.concatenate([h_r[...], am], axis=-1)
        g = _silu(jnp.dot(nm, wn1_r[...], preferred_element_type=_f32) + bn1_r[...])
        h2_r[...] = h_r[...] + jnp.dot(g, wn2_r[...], preferred_element_type=_f32) + bn2_r[...]
        x2_r[...] = x_r[...] + ax / _MAX_IN_DEG

    row = pl.BlockSpec((_BN, _H), lambda i: (i, 0))
    xrow = pl.BlockSpec((_BN, _XP), lambda i: (i, 0))
    amrow = pl.BlockSpec((_NC, _BN, _H), lambda i: (0, i, 0))
    axrow = pl.BlockSpec((_NC, _BN, _XP), lambda i: (0, i, 0))
    return pl.pallas_call(
        body,
        grid=(_N // _BN,),
        in_specs=[row, xrow, amrow, axrow, amrow, axrow, _full2((2 * _H, _H)),
                  _full2((1, _H)), _full2((_H, _H)), _full2((1, _H))],
        out_specs=[row, xrow],
        out_shape=[jax.ShapeDtypeStruct((_N, _H), _f32),
                   jax.ShapeDtypeStruct((_N, _XP), _f32)],
    )(h, x, ag1m, ag1x, ag2m, ag2x, Wn1l, bn1l.reshape(1, _H), Wn2l,
      bn2l.reshape(1, _H))


def _tc_node_last(h, x, ag1m, ag1x, ag2m, ag2x, Wn1l, bn1l, Wn2l, bn2l,
                  Wout, b_out):
    def body(h_r, x_r, a1m_r, a1x_r, a2m_r, a2x_r, wn1_r, bn1_r, wn2_r, bn2_r,
             wo_r, bo_r, o_r, x2_r):
        am = a1m_r[0] + a1m_r[1] + a2m_r[0] + a2m_r[1]
        ax = a1x_r[0] + a1x_r[1] + a2x_r[0] + a2x_r[1]
        nm = jnp.concatenate([h_r[...], am], axis=-1)
        g = _silu(jnp.dot(nm, wn1_r[...], preferred_element_type=_f32) + bn1_r[...])
        h2 = h_r[...] + jnp.dot(g, wn2_r[...], preferred_element_type=_f32) + bn2_r[...]
        o_r[...] = jnp.dot(h2, wo_r[...], preferred_element_type=_f32) + bo_r[...]
        x2_r[...] = x_r[...] + ax / _MAX_IN_DEG

    row = pl.BlockSpec((_BN, _H), lambda i: (i, 0))
    xrow = pl.BlockSpec((_BN, _XP), lambda i: (i, 0))
    amrow = pl.BlockSpec((_NC, _BN, _H), lambda i: (0, i, 0))
    axrow = pl.BlockSpec((_NC, _BN, _XP), lambda i: (0, i, 0))
    return pl.pallas_call(
        body,
        grid=(_N // _BN,),
        in_specs=[row, xrow, amrow, axrow, amrow, axrow, _full2((2 * _H, _H)),
                  _full2((1, _H)), _full2((_H, _H)), _full2((1, _H)),
                  _full2((_H, _H)), _full2((1, _H))],
        out_specs=[row, xrow],
        out_shape=[jax.ShapeDtypeStruct((_N, _H), _f32),
                   jax.ShapeDtypeStruct((_N, _XP), _f32)],
    )(h, x, ag1m, ag1x, ag2m, ag2x, Wn1l, bn1l.reshape(1, _H), Wn2l,
      bn2l.reshape(1, _H), Wout, b_out.reshape(1, _H))


def kernel(feat, coordinate, edge_index, Win, b_in, Wout, b_out,
           We1, be1, We2, be2, Wc1, bc1, Wc2, bc2, Wn1, bn1, Wn2, bn2):
    src1, src2 = _half_edges(edge_index[0])
    dst1, dst2 = _half_edges(edge_index[1])
    x = jnp.pad(coordinate, ((0, 0), (0, _XP - 3)))
    nch = (_E // 2) // _C

    h = _tc_embed(feat, Win, b_in)
    out = None
    for l in range(_DEPTH):
        wl = (We1[l], be1[l], We2[l], be2[l], Wc1[l], bc1[l], Wc2[l], bc2[l])
        hd1, hs1, xs1, xd1 = _sc_gather(h, x, src1, dst1, nch)
        m1, v1 = _tc_edge(hd1, hs1, xs1, xd1, *wl)
        hd2, hs2, xs2, xd2 = _sc_gather(h, x, src2, dst2, nch)
        ag1m, ag1x = _sc_scatter(m1, v1, dst1, nch)
        m2, v2 = _tc_edge(hd2, hs2, xs2, xd2, *wl)
        ag2m, ag2x = _sc_scatter(m2, v2, dst2, nch)
        if l < _DEPTH - 1:
            h, x = _tc_node(h, x, ag1m, ag1x, ag2m, ag2x,
                            Wn1[l], bn1[l], Wn2[l], bn2[l])
        else:
            out, x = _tc_node_last(h, x, ag1m, ag1x, ag2m, ag2x,
                                   Wn1[l], bn1[l], Wn2[l], bn2[l],
                                   Wout, b_out)
    return (out, x[:, :3])
